# baseline (speedup 1.0000x reference)
import functools

import jax
import jax.numpy as jnp
from jax import lax
from jax.experimental import pallas as pl
from jax.experimental.pallas import tpu as pltpu
from jax.experimental.pallas import tpu_sc as plsc

DIM = 64
B = 16384
L = 50
BL = B * L
NC, NS = 2, 16
NW = NC * NS
RPW = BL // NW
SUB = 128
CHUNK = 1024
NSUB = CHUNK // SUB
NCH = RPW // CHUNK
IDX_ROWS = BL // SUB

_sc_mesh = plsc.VectorSubcoreMesh(core_axis_name="c", subcore_axis_name="s")


@functools.partial(
    pl.kernel,
    out_type=jax.ShapeDtypeStruct((2 * BL, DIM), jnp.float32),
    mesh=_sc_mesh,
    scratch_types=[
        pltpu.VMEM((CHUNK,), jnp.int32),
        pltpu.VMEM((CHUNK, DIM), jnp.float32),
        pltpu.SemaphoreType.DMA,
    ],
    compiler_params=pltpu.CompilerParams(use_tc_tiling_on_sc=False),
)
def _sc_gather(semb, temb, xidx, yidx, out, idx_v, rows_v, sem):
    wid = lax.axis_index("s") * NC + lax.axis_index("c")

    def do_side(table, idx_hbm, out_base):
        base = wid * RPW

        def chunk_body(c, carry):
            off = base + c * CHUNK
            pltpu.sync_copy(idx_hbm.at[pl.ds(off, CHUNK)], idx_v)
            descs = []
            for j in range(NSUB):
                descs.append(
                    pltpu.async_copy(
                        table.at[idx_v.at[pl.ds(j * SUB, SUB)]],
                        rows_v.at[pl.ds(j * SUB, SUB)],
                        sem,
                    )
                )
            for d in descs:
                d.wait()
            pltpu.sync_copy(rows_v, out.at[pl.ds(out_base + off, CHUNK)])
            return carry

        lax.fori_loop(0, NCH, chunk_body, 0)

    do_side(semb, xidx, 0)
    do_side(temb, yidx, BL)


_MM_B = 256


def _mm_body(x_ref, w_ref, o_ref):
    y = lax.dot_general(
        x_ref[...], w_ref[...], (((1,), (1,)), ((), ())),
        preferred_element_type=jnp.float32,
    )
    o_ref[...] = y.reshape(1, _MM_B, L, DIM)


def _project(gathered, w):
    rows = _MM_B * L
    return pl.pallas_call(
        _mm_body,
        grid=(2, B // _MM_B),
        in_specs=[
            pl.BlockSpec((rows, DIM), lambda s, i: (s * (B // _MM_B) + i, 0)),
            pl.BlockSpec((DIM, DIM), lambda s, i: (0, 0)),
        ],
        out_specs=pl.BlockSpec((1, _MM_B, L, DIM), lambda s, i: (s, i, 0, 0)),
        out_shape=jax.ShapeDtypeStruct((2, B, L, DIM), jnp.float32),
    )(gathered, w)


def kernel(W_m, semb_table, temb_table, x_idx, y_idx):
    xr = x_idx.astype(jnp.int32).reshape(BL)
    yr = y_idx.astype(jnp.int32).reshape(BL)
    gathered = _sc_gather(semb_table, temb_table, xr, yr)
    return _project(gathered, W_m)


# speedup vs baseline: 1.1927x; 1.1927x over previous
import functools

import jax
import jax.numpy as jnp
from jax import lax
from jax.experimental import pallas as pl
from jax.experimental.pallas import tpu as pltpu
from jax.experimental.pallas import tpu_sc as plsc

DIM = 64
B = 16384
L = 50
BL = B * L
NC, NS = 2, 16
NW = NC * NS
RPW = BL // NW
SUB = 128
CHUNK = 1024
NSUB = CHUNK // SUB
NCH = RPW // CHUNK
IDX_ROWS = BL // SUB

_sc_mesh = plsc.VectorSubcoreMesh(core_axis_name="c", subcore_axis_name="s")


@functools.partial(
    pl.kernel,
    out_type=jax.ShapeDtypeStruct((2 * BL, DIM), jnp.float32),
    mesh=_sc_mesh,
    scratch_types=[
        pltpu.VMEM((CHUNK,), jnp.int32),
        pltpu.VMEM((CHUNK, DIM), jnp.float32),
        pltpu.SemaphoreType.DMA,
    ],
    compiler_params=pltpu.CompilerParams(use_tc_tiling_on_sc=False),
)
def _sc_gather(semb, temb, xidx, yidx, out, idx_v, rows_v, sem):
    wid = lax.axis_index("s") * NC + lax.axis_index("c")

    def do_side(table, idx_hbm, out_base):
        base = wid * RPW

        def chunk_body(c, carry):
            off = base + c * CHUNK
            pltpu.sync_copy(idx_hbm.at[pl.ds(off, CHUNK)], idx_v)
            descs = []
            for j in range(NSUB):
                descs.append(
                    pltpu.async_copy(
                        table.at[idx_v.at[pl.ds(j * SUB, SUB)]],
                        rows_v.at[pl.ds(j * SUB, SUB)],
                        sem,
                    )
                )
            for d in descs:
                d.wait()
            pltpu.sync_copy(rows_v, out.at[pl.ds(out_base + off, CHUNK)])
            return carry

        lax.fori_loop(0, NCH, chunk_body, 0)

    do_side(semb, xidx, 0)
    do_side(temb, yidx, BL)


_MM_ROWS = 12800


def _mm_body(x_ref, w_ref, o_ref):
    o_ref[...] = lax.dot_general(
        x_ref[...], w_ref[...], (((1,), (0,)), ((), ())),
        preferred_element_type=jnp.float32,
    )


def _project(gathered2, w2):
    return pl.pallas_call(
        _mm_body,
        grid=(BL // _MM_ROWS,),
        in_specs=[
            pl.BlockSpec((_MM_ROWS, 2 * DIM), lambda i: (i, 0)),
            pl.BlockSpec((2 * DIM, 2 * DIM), lambda i: (0, 0)),
        ],
        out_specs=pl.BlockSpec((_MM_ROWS, 2 * DIM), lambda i: (i, 0)),
        out_shape=jax.ShapeDtypeStruct((BL, 2 * DIM), jnp.float32),
    )(gathered2, w2)


def kernel(W_m, semb_table, temb_table, x_idx, y_idx):
    xr = x_idx.astype(jnp.int32).reshape(BL)
    yr = y_idx.astype(jnp.int32).reshape(BL)
    gathered = _sc_gather(semb_table, temb_table, xr, yr)
    wt = W_m.T
    z = jnp.zeros((DIM, DIM), jnp.float32)
    w2 = jnp.block([[wt, z], [z, wt]])
    proj = _project(gathered.reshape(BL, 2 * DIM), w2)
    return proj.reshape(2, B, L, DIM)
